# Initial kernel scaffold; baseline (speedup 1.0000x reference)
import jax
import jax.numpy as jnp
from jax.experimental import pallas as pl


def kernel(x, edge_index, mask, mask_idx, W1, b1, g1, be1, W2, b2, g2, be2):
    raise NotImplementedError("write your pallas kernel here")


# speedup vs baseline: 11.5187x; 11.5187x over previous
import functools

import jax
import jax.numpy as jnp
from jax import lax
from jax.experimental import pallas as pl
from jax.experimental.pallas import tpu as pltpu
from jax.experimental.pallas import tpu_sc as plsc

N = 10000
E = 160000
D = 256
H = 128
EPS = 1e-5

NC = 2
NS = 16
L = 16

N_PAD = 10240
SL = N_PAD // NS
EPT = E // NS
CH = 48
NCH = -(-EPT // CH)
EPTP = NCH * CH

_i32 = jnp.int32
_f32 = jnp.float32


def _newton_rsqrt(x):
    bits = plsc.bitcast(x, _i32)
    seed = jnp.int32(0x5F3759DF) - lax.shift_right_logical(bits, 1)
    y = plsc.bitcast(seed, _f32)
    for _ in range(3):
        y = y * (1.5 - 0.5 * x * y * y)
    return y


def _iota16():
    return lax.iota(_i32, 16)


def _precompute_body(m_hbm, src_hbm, dst_hbm, w_hbm,
                     m_v, dk_v, src_v, dst_v, w_v, acc_s, tmp_s,
                     shared_deg, shared_dk):
    c = lax.axis_index("c")
    s = lax.axis_index("s")

    pltpu.sync_copy(m_hbm, m_v)
    pltpu.sync_copy(src_hbm.at[s], src_v)
    pltpu.sync_copy(dst_hbm.at[s], dst_v)

    @pl.loop(0, N_PAD // L)
    def _keep(i):
        sl = pl.ds(i * L, L)
        v = m_v[sl]
        m_v[sl] = jnp.where(v > 0.0, 1.0, 0.0).astype(_f32)
        dk_v[sl] = jnp.zeros((L,), _f32)

    @pl.loop(0, NCH)
    def _deg(j):
        base = j * CH
        for g in range(CH // L):
            sl = pl.ds(base + g * L, L)
            sidx = src_v[sl]
            didx = dst_v[sl]
            ew = plsc.load_gather(m_v, [sidx]) * plsc.load_gather(m_v, [didx])
            valid = (base + g * L + _iota16()) < EPT
            ew = jnp.where(valid, ew, 0.0)
            for k in range(L):
                plsc.addupdate_scatter(dk_v, [sidx], ew, mask=_iota16() == k)

    pltpu.sync_copy(dk_v, shared_deg.at[s])
    plsc.subcore_barrier()

    @pl.loop(0, SL // L)
    def _zacc(i):
        acc_s[pl.ds(i * L, L)] = jnp.zeros((L,), _f32)

    @pl.loop(0, NS)
    def _red(t):
        pltpu.sync_copy(shared_deg.at[t, pl.ds(s * SL, SL)], tmp_s)

        @pl.loop(0, SL // L)
        def _add(i):
            sl = pl.ds(i * L, L)
            acc_s[sl] = acc_s[sl] + tmp_s[sl]

    @pl.loop(0, SL // L)
    def _dk(i):
        sl = pl.ds(i * L, L)
        deg = acc_s[sl]
        keep = m_v[pl.ds(s * SL + i * L, L)]
        dis = jnp.where(deg > 0.0, _newton_rsqrt(deg), 0.0)
        acc_s[sl] = dis * keep

    pltpu.sync_copy(acc_s, shared_dk.at[pl.ds(s * SL, SL)])
    plsc.subcore_barrier()
    pltpu.sync_copy(shared_dk, dk_v)

    @pl.loop(0, NCH)
    def _w(j):
        base = j * CH
        for g in range(CH // L):
            sl = pl.ds(base + g * L, L)
            sidx = src_v[sl]
            didx = dst_v[sl]
            wv = -(plsc.load_gather(dk_v, [sidx]) * plsc.load_gather(dk_v, [didx]))
            valid = (base + g * L + _iota16()) < EPT
            w_v[sl] = jnp.where(valid, wv, 0.0)

    @pl.when(c == 0)
    def _store():
        pltpu.sync_copy(w_v, w_hbm.at[s])


def _precompute(m_pad, src_flat, dst_flat):
    mesh = plsc.VectorSubcoreMesh(core_axis_name="c", subcore_axis_name="s")
    kern = pl.kernel(
        _precompute_body,
        out_type=jax.ShapeDtypeStruct((NS, EPTP), _f32),
        mesh=mesh,
        compiler_params=pltpu.CompilerParams(needs_layout_passes=False),
        scratch_types=[
            pltpu.VMEM((N_PAD,), _f32),
            pltpu.VMEM((N_PAD,), _f32),
            pltpu.VMEM((EPTP,), _i32),
            pltpu.VMEM((EPTP,), _i32),
            pltpu.VMEM((EPTP,), _f32),
            pltpu.VMEM((SL,), _f32),
            pltpu.VMEM((SL,), _f32),
            pltpu.VMEM_SHARED((NS, N_PAD), _f32),
            pltpu.VMEM_SHARED((N_PAD,), _f32),
        ],
    )
    return kern(m_pad, src_flat, dst_flat)


PD = 5


def _prop2_body(t_hbm, src_hbm, dw_hbm, zeros_hbm, tx1_hbm, p2_hbm,
                src_v, dw0, dw1, dw2, dw3, dw4,
                rows0, rows1, rows2, rows3, rows4, acc_sh, *sems):
    c = lax.axis_index("c")
    s = lax.axis_index("s")
    myrows = pl.ds(s * SL, SL)

    pltpu.sync_copy(src_hbm.at[s], src_v)

    bufs = (rows0, rows1, rows2, rows3, rows4)
    dwbufs = (dw0, dw1, dw2, dw3, dw4)
    gsems = sems[0:PD]
    dsems = sems[PD:2 * PD]
    ssems = sems[2 * PD:3 * PD]

    def start_loads(p, j, b):
        idx = src_v.at[pl.ds(j * CH, CH)]

        @pl.when(p == 0)
        def _g0():
            pltpu.async_copy(t_hbm.at[c].at[idx], bufs[b], gsems[b])

        @pl.when(p == 1)
        def _g1():
            pltpu.async_copy(tx1_hbm.at[c].at[idx], bufs[b], gsems[b])

        pltpu.async_copy(dw_hbm.at[s, j], dwbufs[b], dsems[b])

    def wait_loads(j, b):
        idx = src_v.at[pl.ds(j * CH, CH)]
        pltpu.make_async_copy(t_hbm.at[c].at[idx], bufs[b], gsems[b]).wait()
        pltpu.make_async_copy(dw_hbm.at[s, j], dwbufs[b], dsems[b]).wait()

    def scatter_desc(j, b):
        return pltpu.make_async_copy(bufs[b], acc_sh.at[dwbufs[b].at[0]],
                                     ssems[b])

    def scale(b):
        rows = bufs[b]
        for rg in range(CH // L):
            w16 = plsc.bitcast(dwbufs[b][1, pl.ds(rg * L, L)], _f32)
            for ri in range(L):
                r = rg * L + ri
                wrow = jnp.take(w16, jnp.full((L,), ri, _i32))
                for q in range(H // L):
                    sl = pl.ds(q * L, L)
                    rows[r, sl] = rows[r, sl] * wrow

    @pl.loop(0, 2)
    def _pass(p):
        pltpu.sync_copy(zeros_hbm.at[myrows], acc_sh.at[myrows])
        plsc.subcore_barrier()

        for k in range(PD - 1):
            if k < NCH:
                start_loads(p, k, k)

        @pl.loop(0, NCH + PD - 1, step=PD)
        def _chunk(jo):
            for b in range(PD):
                j = jo + b

                @pl.when(j < NCH)
                def _do():
                    wait_loads(j, b)
                    scale(b)

                    @pl.when(j + PD - 1 < NCH)
                    def _next():
                        @pl.when(j >= 1)
                        def _drain():
                            scatter_desc(j - 1, (b - 1) % PD).wait()
                        start_loads(p, j + PD - 1, (b - 1) % PD)

                    pltpu.async_copy(bufs[b], acc_sh.at[dwbufs[b].at[0]],
                                     ssems[b], add=True)

        for k in range(min(PD, NCH)):
            j = NCH - 1 - k
            scatter_desc(j, j % PD).wait()

        plsc.subcore_barrier()

        @pl.when(p == 0)
        def _out0():
            pltpu.sync_copy(acc_sh.at[myrows], tx1_hbm.at[c, myrows])

        @pl.when(p == 1)
        def _out1():
            pltpu.sync_copy(acc_sh.at[myrows], p2_hbm.at[c, myrows])

        plsc.subcore_barrier()


def _prop2(t, src_flat, dw, zeros_acc):
    mesh = plsc.VectorSubcoreMesh(core_axis_name="c", subcore_axis_name="s")
    kern = pl.kernel(
        _prop2_body,
        out_type=(jax.ShapeDtypeStruct((NC, N_PAD, H), _f32),
                  jax.ShapeDtypeStruct((NC, N_PAD, H), _f32)),
        mesh=mesh,
        compiler_params=pltpu.CompilerParams(needs_layout_passes=False),
        scratch_types=(
            [pltpu.VMEM((EPTP,), _i32)]
            + [pltpu.VMEM((2, CH), _i32) for _ in range(PD)]
            + [pltpu.VMEM((CH, H), _f32) for _ in range(PD)]
            + [pltpu.VMEM_SHARED((N_PAD, H), _f32)]
            + [pltpu.SemaphoreType.DMA] * (3 * PD)
        ),
    )
    return kern(t, src_flat, dw, zeros_acc)


def _mask_body(x_ref, m_ref, o_ref):
    prod = x_ref[...] * m_ref[...]
    o_ref[0] = prod[:, :H]
    o_ref[1] = prod[:, H:]


def _mask_kernel(x_pad, m_col):
    bn = 1024
    grid = N_PAD // bn
    return pl.pallas_call(
        _mask_body,
        grid=(grid,),
        in_specs=[
            pl.BlockSpec((bn, D), lambda i: (i, 0)),
            pl.BlockSpec((bn, 1), lambda i: (i, 0)),
        ],
        out_specs=pl.BlockSpec((NC, bn, H), lambda i: (0, i, 0)),
        out_shape=jax.ShapeDtypeStruct((NC, N_PAD, H), _f32),
    )(x_pad, m_col)


def _dense_body(final, tx0_ref, tx1_ref, p2_ref, a_ref, b_ref, c_ref,
                bias_ref, g_ref, be_ref, o_ref):
    h = bias_ref[...].astype(_f32)
    h = h + jnp.dot(tx0_ref[0], a_ref[0], preferred_element_type=_f32)
    h = h + jnp.dot(tx0_ref[1], a_ref[1], preferred_element_type=_f32)
    h = h + jnp.dot(tx1_ref[0], b_ref[0], preferred_element_type=_f32)
    h = h + jnp.dot(tx1_ref[1], b_ref[1], preferred_element_type=_f32)
    h = h + jnp.dot(p2_ref[0], c_ref[0], preferred_element_type=_f32)
    h = h + jnp.dot(p2_ref[1], c_ref[1], preferred_element_type=_f32)
    mu = jnp.mean(h, axis=1, keepdims=True)
    d = h - mu
    var = jnp.mean(d * d, axis=1, keepdims=True)
    o = d * lax.rsqrt(var + EPS) * g_ref[...] + be_ref[...]
    if final:
        o_ref[...] = o
    else:
        o_ref[0] = o[:, :H]
        o_ref[1] = o[:, H:]


def _dense(tx0, tx1, p2, a, b, c, bias, g, be, final):
    bn = 1024
    grid = N_PAD // bn
    plane = pl.BlockSpec((NC, bn, H), lambda i: (0, i, 0))
    wspec = pl.BlockSpec((NC, H, D), lambda i: (0, 0, 0))
    vspec = pl.BlockSpec((1, D), lambda i: (0, 0))
    if final:
        out_specs = pl.BlockSpec((bn, D), lambda i: (i, 0))
        out_shape = jax.ShapeDtypeStruct((N_PAD, D), _f32)
    else:
        out_specs = plane
        out_shape = jax.ShapeDtypeStruct((NC, N_PAD, H), _f32)
    return pl.pallas_call(
        functools.partial(_dense_body, final),
        grid=(grid,),
        in_specs=[plane, plane, plane, wspec, wspec, wspec, vspec, vspec, vspec],
        out_specs=out_specs,
        out_shape=out_shape,
    )(tx0, tx1, p2, a, b, c, bias, g, be)


def kernel(x, edge_index, mask, mask_idx, W1, b1, g1, be1, W2, b2, g2, be2):
    m = jnp.take(mask, mask_idx, axis=1).astype(_f32)
    m_pad = jnp.pad(m, (0, N_PAD - N))
    x_pad = jnp.pad(x, ((0, N_PAD - N), (0, 0)))

    src = edge_index[0]
    dst = edge_index[1]
    src_t = jnp.pad(src.reshape(NS, EPT), ((0, 0), (0, EPTP - EPT)))
    dst_t = jnp.pad(dst.reshape(NS, EPT), ((0, 0), (0, EPTP - EPT)))

    zeros_acc = jnp.zeros((N_PAD, H), _f32)

    w = _precompute(m_pad, src_t, dst_t)
    dw = jnp.stack([dst_t.reshape(NS, NCH, CH),
                    lax.bitcast_convert_type(w, _i32).reshape(NS, NCH, CH)],
                   axis=2)

    h = _mask_kernel(x_pad, m_pad.reshape(N_PAD, 1))

    for (W, b, g, be, final) in ((W1, b1, g1, be1, False),
                                 (W2, b2, g2, be2, True)):
        a_w = (W[0] - W[2]).reshape(NC, H, D)
        b_w = W[1].reshape(NC, H, D)
        c_w = (2.0 * W[2]).reshape(NC, H, D)
        tx1, p2 = _prop2(h, src_t, dw, zeros_acc)
        h = _dense(h, tx1, p2, a_w, b_w, c_w,
                   b.reshape(1, D), g.reshape(1, D), be.reshape(1, D), final)
    return h[:N]
